# baseline (speedup 1.0000x reference)
import jax
import jax.numpy as jnp
from jax.experimental import pallas as pl
from jax.experimental.pallas import tpu as pltpu

_SEQ_BLOCK = 1024
_NSLOTS = 2


def _body(x_ref, pos_ref, out_hbm, scratch, sems):
    i = pl.program_id(0)
    n = pl.num_programs(0)
    blk = x_ref.shape[1]
    slot = jax.lax.rem(i, _NSLOTS)

    @pl.when(i >= _NSLOTS)
    def _wait_prev():
        pltpu.make_async_copy(
            scratch.at[slot],
            out_hbm.at[:, pl.ds((i - _NSLOTS) * blk, blk), :],
            sems.at[slot],
        ).wait()

    scratch[slot] = x_ref[...] + pos_ref[None]
    pltpu.make_async_copy(
        scratch.at[slot],
        out_hbm.at[:, pl.ds(i * blk, blk), :],
        sems.at[slot],
    ).start()

    @pl.when(i == n - 1)
    def _drain_all():
        for s in range(_NSLOTS):
            step = n - _NSLOTS + s
            pltpu.make_async_copy(
                scratch.at[jax.lax.rem(jnp.int32(step), _NSLOTS)],
                out_hbm.at[:, pl.ds(step * blk, blk), :],
                sems.at[jax.lax.rem(jnp.int32(step), _NSLOTS)],
            ).wait()


def kernel(x, pos_embedding):
    batch, seq, d = x.shape
    pos = pos_embedding[:seq]
    blk = min(_SEQ_BLOCK, seq)
    grid = (seq // blk,)
    return pl.pallas_call(
        _body,
        grid=grid,
        in_specs=[
            pl.BlockSpec((batch, blk, d), lambda i: (0, i, 0)),
            pl.BlockSpec((blk, d), lambda i: (i, 0)),
        ],
        out_specs=pl.BlockSpec(memory_space=pl.ANY),
        out_shape=jax.ShapeDtypeStruct((batch, seq, d), x.dtype),
        scratch_shapes=[
            pltpu.VMEM((_NSLOTS, batch, blk, d), x.dtype),
            pltpu.SemaphoreType.DMA((_NSLOTS,)),
        ],
    )(x, pos)


# speedup vs baseline: 1.0042x; 1.0042x over previous
import jax
import jax.numpy as jnp
from jax.experimental import pallas as pl
from jax.experimental.pallas import tpu as pltpu

_SEQ_BLOCK = 512
_NSLOTS = 4


def _body(x_ref, pos_ref, out_hbm, scratch, sems):
    i = pl.program_id(0)
    n = pl.num_programs(0)
    blk = x_ref.shape[1]
    slot = jax.lax.rem(i, _NSLOTS)

    @pl.when(i >= _NSLOTS)
    def _wait_prev():
        pltpu.make_async_copy(
            scratch.at[slot],
            out_hbm.at[:, pl.ds((i - _NSLOTS) * blk, blk), :],
            sems.at[slot],
        ).wait()

    scratch[slot] = x_ref[...] + pos_ref[None]
    pltpu.make_async_copy(
        scratch.at[slot],
        out_hbm.at[:, pl.ds(i * blk, blk), :],
        sems.at[slot],
    ).start()

    @pl.when(i == n - 1)
    def _drain_all():
        for s in range(_NSLOTS):
            step = n - _NSLOTS + s
            pltpu.make_async_copy(
                scratch.at[jax.lax.rem(jnp.int32(step), _NSLOTS)],
                out_hbm.at[:, pl.ds(step * blk, blk), :],
                sems.at[jax.lax.rem(jnp.int32(step), _NSLOTS)],
            ).wait()


def kernel(x, pos_embedding):
    batch, seq, d = x.shape
    pos = pos_embedding[:seq]
    blk = min(_SEQ_BLOCK, seq)
    grid = (seq // blk,)
    return pl.pallas_call(
        _body,
        grid=grid,
        in_specs=[
            pl.BlockSpec((batch, blk, d), lambda i: (0, i, 0)),
            pl.BlockSpec((blk, d), lambda i: (i, 0)),
        ],
        out_specs=pl.BlockSpec(memory_space=pl.ANY),
        out_shape=jax.ShapeDtypeStruct((batch, seq, d), x.dtype),
        scratch_shapes=[
            pltpu.VMEM((_NSLOTS, batch, blk, d), x.dtype),
            pltpu.SemaphoreType.DMA((_NSLOTS,)),
        ],
    )(x, pos)
